# baseline (speedup 1.0000x reference)
import functools
import math

import jax
import jax.numpy as jnp
from jax import lax
from jax.experimental import pallas as pl
from jax.experimental.pallas import tpu as pltpu
from jax.experimental.pallas import tpu_sc as plsc

D = 64
SCALE = math.sqrt(D)
NC, NS = 2, 16
NW = NC * NS

CHUNK = 1024
IDX_W = 128
IDX_ROWS = CHUNK // IDX_W


def _pack_idx_body(x_ref, o_ref):
    o_ref[...] = x_ref[...].reshape(o_ref.shape)


def _gather_body(n_chunks, b_per_w, x_hbm, table_hbm, out_hbm,
                 idx_v, rows_v, sem):
    wid = lax.axis_index("s") * NC + lax.axis_index("c")
    base = wid * b_per_w

    def chunk_body(g, carry):
        cbase = base + g * CHUNK
        irow = pl.multiple_of(cbase // IDX_W, 8)
        pltpu.sync_copy(x_hbm.at[pl.ds(irow, IDX_ROWS)], idx_v)
        cps = [
            pltpu.async_copy(table_hbm.at[idx_v.at[j]],
                             rows_v.at[pl.ds(j * IDX_W, IDX_W)], sem)
            for j in range(IDX_ROWS)
        ]
        for cp in cps:
            cp.wait()
        pltpu.sync_copy(rows_v, out_hbm.at[pl.ds(cbase, CHUNK)])
        return carry

    lax.fori_loop(0, n_chunks, chunk_body, 0)


def _scale_fmt_body(bb, h, x_ref, o_ref):
    v = x_ref[...].reshape(bb, h // 2, 2 * D)
    even = v[:, :, :D] * SCALE
    odd = v[:, :, D:] * SCALE
    w = jnp.stack((even, odd), axis=2)
    o_ref[...] = w.reshape(bb, h, D)


def kernel(x, table):
    b, h = x.shape
    n = b * h
    assert n % (NW * CHUNK) == 0 and n % IDX_W == 0
    b_per_w = n // NW
    n_chunks = b_per_w // CHUNK

    packed = x.reshape(n // IDX_W, IDX_W)

    mesh = plsc.VectorSubcoreMesh(core_axis_name="c", subcore_axis_name="s")
    raw = pl.kernel(
        functools.partial(_gather_body, n_chunks, b_per_w),
        mesh=mesh,
        compiler_params=pltpu.CompilerParams(use_tc_tiling_on_sc=False),
        out_type=jax.ShapeDtypeStruct((n, D), jnp.float32),
        scratch_types=[
            pltpu.VMEM((IDX_ROWS, IDX_W), jnp.int32),
            pltpu.VMEM((CHUNK, D), jnp.float32),
            pltpu.SemaphoreType.DMA,
        ],
    )(packed, table)

    raw2 = raw.reshape(n * D // 128, 128)

    bb = 128
    rows_per_bb = bb * h * D // 128
    out = pl.pallas_call(
        functools.partial(_scale_fmt_body, bb, h),
        grid=(b // bb,),
        in_specs=[pl.BlockSpec((rows_per_bb, 128), lambda i: (i, 0))],
        out_specs=pl.BlockSpec((bb, h, D), lambda i: (i, 0, 0)),
        out_shape=jax.ShapeDtypeStruct((b, h, D), jnp.float32),
    )(raw2)
    return out


# speedup vs baseline: 1.2927x; 1.2927x over previous
import functools
import math

import jax
import jax.numpy as jnp
from jax import lax
from jax.experimental import pallas as pl
from jax.experimental.pallas import tpu as pltpu
from jax.experimental.pallas import tpu_sc as plsc

D = 64
DP = 128
SCALE = math.sqrt(D)
NC, NS = 2, 16
NW = NC * NS

CHUNK = 512
IDX_W = 128
IDX_ROWS = CHUNK // IDX_W
STAGE = 2 * CHUNK


def _gather_body(n_stages, b_per_w, x_hbm, table_hbm, out_hbm,
                 idx_v, rows_v, sem):
    wid = lax.axis_index("s") * NC + lax.axis_index("c")
    base = wid * b_per_w

    def stage_body(g, carry):
        sbase = base + g * STAGE
        irow = pl.multiple_of(sbase // IDX_W, 8)
        pltpu.sync_copy(x_hbm.at[pl.ds(irow, STAGE // IDX_W)], idx_v)
        for half in range(2):
            cps = [
                pltpu.async_copy(
                    table_hbm.at[idx_v.at[half * IDX_ROWS + j]],
                    rows_v.at[pl.ds(j * IDX_W, IDX_W)], sem)
                for j in range(IDX_ROWS)
            ]
            for cp in cps:
                cp.wait()
            pltpu.sync_copy(
                rows_v, out_hbm.at[pl.ds(sbase + half * CHUNK, CHUNK)])
        return carry

    lax.fori_loop(0, n_stages, stage_body, 0)


def _scale_tr_body(bb, h, x_ref, o_ref):
    v3 = x_ref[...].reshape(bb, h, DP)
    for hh in range(h):
        v = v3[:, hh, :D]
        o_ref[hh] = jnp.transpose(v) * SCALE


def kernel(x, table):
    b, h = x.shape
    n = b * h
    assert n % (NW * STAGE) == 0
    b_per_w = n // NW
    n_stages = b_per_w // STAGE

    x_flat = x.reshape(n // IDX_W, IDX_W)

    v = table.shape[0]
    tabp = jnp.concatenate(
        [table, jnp.zeros((v, DP - D), jnp.float32)], axis=1)

    mesh = plsc.VectorSubcoreMesh(core_axis_name="c", subcore_axis_name="s")
    raw = pl.kernel(
        functools.partial(_gather_body, n_stages, b_per_w),
        mesh=mesh,
        compiler_params=pltpu.CompilerParams(use_tc_tiling_on_sc=False),
        out_type=jax.ShapeDtypeStruct((n, DP), jnp.float32),
        scratch_types=[
            pltpu.VMEM((STAGE // IDX_W, IDX_W), jnp.int32),
            pltpu.VMEM((CHUNK, DP), jnp.float32),
            pltpu.SemaphoreType.DMA,
        ],
    )(x_flat, tabp)

    bb = 256
    out_t = pl.pallas_call(
        functools.partial(_scale_tr_body, bb, h),
        grid=(b // bb,),
        in_specs=[pl.BlockSpec((bb * h, DP), lambda bi: (bi, 0))],
        out_specs=pl.BlockSpec((h, D, bb), lambda bi: (0, 0, bi)),
        out_shape=jax.ShapeDtypeStruct((h, D, b), jnp.float32),
    )(raw)
    return jnp.transpose(out_t, (2, 0, 1))


# speedup vs baseline: 1.5296x; 1.1832x over previous
import functools
import math

import jax
import jax.numpy as jnp
from jax import lax
from jax.experimental import pallas as pl
from jax.experimental.pallas import tpu as pltpu
from jax.experimental.pallas import tpu_sc as plsc

D = 64
DP = 128
SCALE = math.sqrt(D)
NC, NS = 2, 16
NW = NC * NS

CHUNK = 512
IDX_W = 128
IDX_ROWS = CHUNK // IDX_W
STAGE = 2 * CHUNK


def _gather_body(n_stages, b_per_w, x_hbm, table_hbm, out_hbm,
                 idx_v, rows_v, sem):
    wid = lax.axis_index("s") * NC + lax.axis_index("c")
    base = wid * b_per_w

    def stage_body(g, carry):
        sbase = base + g * STAGE
        irow = pl.multiple_of(sbase // IDX_W, 8)
        pltpu.sync_copy(x_hbm.at[pl.ds(irow, STAGE // IDX_W)], idx_v)
        for half in range(2):
            cps = [
                pltpu.async_copy(
                    table_hbm.at[idx_v.at[half * IDX_ROWS + j]],
                    rows_v.at[pl.ds(j * IDX_W, IDX_W)], sem)
                for j in range(IDX_ROWS)
            ]
            for cp in cps:
                cp.wait()
            pltpu.sync_copy(
                rows_v, out_hbm.at[pl.ds(sbase + half * CHUNK, CHUNK)])
        return carry

    lax.fori_loop(0, n_stages, stage_body, 0)


def _widen_tab_body(cb, x_ref, o_ref):
    o_ref[:, :D] = jnp.transpose(x_ref[...])
    o_ref[:, D:] = jnp.zeros((cb, DP - D), jnp.float32)


def _scale_tr_body(bb, h, x_ref, o_ref):
    v3 = x_ref[...].reshape(bb, h, DP)
    for hh in range(h):
        v = v3[:, hh, :D]
        o_ref[hh] = jnp.transpose(v) * SCALE


def kernel(x, table):
    b, h = x.shape
    n = b * h
    assert n % (NW * STAGE) == 0
    b_per_w = n // NW
    n_stages = b_per_w // STAGE

    x_flat = x.reshape(n // IDX_W, IDX_W)

    v = table.shape[0]
    cb = 4096
    tabp = pl.pallas_call(
        functools.partial(_widen_tab_body, cb),
        grid=(v // cb,),
        in_specs=[pl.BlockSpec((D, cb), lambda i: (0, i))],
        out_specs=pl.BlockSpec((cb, DP), lambda i: (i, 0)),
        out_shape=jax.ShapeDtypeStruct((v, DP), jnp.float32),
    )(table.T)

    mesh = plsc.VectorSubcoreMesh(core_axis_name="c", subcore_axis_name="s")
    raw = pl.kernel(
        functools.partial(_gather_body, n_stages, b_per_w),
        mesh=mesh,
        compiler_params=pltpu.CompilerParams(use_tc_tiling_on_sc=False),
        out_type=jax.ShapeDtypeStruct((n, DP), jnp.float32),
        scratch_types=[
            pltpu.VMEM((STAGE // IDX_W, IDX_W), jnp.int32),
            pltpu.VMEM((CHUNK, DP), jnp.float32),
            pltpu.SemaphoreType.DMA,
        ],
    )(x_flat, tabp)

    bb = 256
    out_t = pl.pallas_call(
        functools.partial(_scale_tr_body, bb, h),
        grid=(b // bb,),
        in_specs=[pl.BlockSpec((bb * h, DP), lambda bi: (bi, 0))],
        out_specs=pl.BlockSpec((h, D, bb), lambda bi: (0, 0, bi)),
        out_shape=jax.ShapeDtypeStruct((h, D, b), jnp.float32),
    )(raw)
    return jnp.transpose(out_t, (2, 0, 1))
